# baseline (speedup 1.0000x reference)
import functools

import jax
import jax.numpy as jnp
from jax import lax
from jax.experimental import pallas as pl
from jax.experimental.pallas import tpu as pltpu
from jax.experimental.pallas import tpu_sc as plsc

_NUM_CODES = 8192
_CODE_DIM = 32
_BETA = 0.25
_B = 8192
_BR = 128
_NB = _B // _BR

_NC = 2
_NS = 16
_NW = _NC * _NS
_RPW = _B // _NW
_GCH = 128
_NCH = _RPW // _GCH


def _vq_body(h_ref, cbt_ref, cb_ref, g_ref, q_ref, ct_ref, idx_ref, lp_ref):
    h = h_ref[...]
    cbt2 = cbt_ref[...]
    cb_sq = 0.25 * jnp.sum(cbt2 * cbt2, axis=0, keepdims=True)
    h_sq = jnp.sum(h * h, axis=1, keepdims=True)
    prod2 = jnp.dot(h, cbt2, preferred_element_type=jnp.float32)
    dist = (h_sq + cb_sq) - prod2
    x = g_ref[...] - dist
    m = jnp.max(x, axis=1, keepdims=True)
    e = jnp.exp(x - m)
    s = jnp.sum(e, axis=1, keepdims=True)
    inv = 1.0 / s
    q = e * inv
    q_ref[...] = q

    cb16 = cb_ref[...].astype(jnp.bfloat16)
    ct = jnp.dot(q.astype(jnp.bfloat16), cb16,
                 preferred_element_type=jnp.float32)
    ct_ref[...] = ct

    iota = jax.lax.broadcasted_iota(jnp.int32, (_BR, _NUM_CODES), 1)
    idx_ref[...] = jnp.min(jnp.where(x == m, iota, _NUM_CODES),
                           axis=1, keepdims=True)
    dsel = jnp.min(jnp.where(x == m, dist, jnp.float32(3.0e38)),
                   axis=1, keepdims=True)
    part = jnp.sum(dsel, axis=(0, 1), keepdims=True)
    lp_ref[...] = jnp.broadcast_to(part[None], (1, 1, 128))


_DPAD = 128


def _sc_gather(cb_hbm, idx_hbm, ch_hbm, cq_hbm, idx_v, rows_v, sem):
    wid = lax.axis_index("s") * _NC + lax.axis_index("c")
    base = wid * _RPW
    pltpu.sync_copy(idx_hbm.at[pl.ds(wid * _NCH, _NCH)], idx_v)
    for j in range(_NCH):
        pltpu.async_copy(cb_hbm.at[idx_v.at[j]],
                         rows_v.at[pl.ds(j * _GCH, _GCH)], sem).wait()
    pltpu.sync_copy(rows_v, ch_hbm.at[pl.ds(base, _RPW)])
    pltpu.sync_copy(rows_v, cq_hbm.at[pl.ds(base, _RPW)])


@jax.jit
def kernel(h, codebook, gumbel):
    cbt2 = 2.0 * codebook.T
    q, ct, idx, lp = pl.pallas_call(
        _vq_body,
        grid=(_NB,),
        in_specs=[
            pl.BlockSpec((_BR, _CODE_DIM), lambda i: (i, 0)),
            pl.BlockSpec((_CODE_DIM, _NUM_CODES), lambda i: (0, 0)),
            pl.BlockSpec((_NUM_CODES, _CODE_DIM), lambda i: (0, 0)),
            pl.BlockSpec((_BR, _NUM_CODES), lambda i: (i, 0)),
        ],
        out_specs=[
            pl.BlockSpec((_BR, _NUM_CODES), lambda i: (i, 0)),
            pl.BlockSpec((_BR, _CODE_DIM), lambda i: (i, 0)),
            pl.BlockSpec((_BR, 1), lambda i: (i, 0)),
            pl.BlockSpec((1, 1, 128), lambda i: (i, 0, 0)),
        ],
        out_shape=[
            jax.ShapeDtypeStruct((_B, _NUM_CODES), jnp.float32),
            jax.ShapeDtypeStruct((_B, _CODE_DIM), jnp.float32),
            jax.ShapeDtypeStruct((_B, 1), jnp.int32),
            jax.ShapeDtypeStruct((_NB, 1, 128), jnp.float32),
        ],
        compiler_params=pltpu.CompilerParams(
            dimension_semantics=("parallel",),
        ),
    )(h, cbt2, codebook, gumbel)

    idx2d = idx.reshape(_NW * _NCH, _GCH)
    cb_pad = jnp.pad(codebook, ((0, 0), (0, _DPAD - _CODE_DIM)))

    mesh = plsc.VectorSubcoreMesh(core_axis_name="c", subcore_axis_name="s")
    sc = pl.kernel(
        _sc_gather,
        mesh=mesh,
        out_type=[
            jax.ShapeDtypeStruct((_B, _DPAD), jnp.float32),
            jax.ShapeDtypeStruct((_B, _DPAD), jnp.float32),
        ],
        scratch_types=[
            pltpu.VMEM((_NCH, _GCH), jnp.int32),
            pltpu.VMEM((_RPW, _DPAD), jnp.float32),
            pltpu.SemaphoreType.DMA,
        ],
    )
    ch_p, cq_p = sc(cb_pad, idx2d)
    ch = ch_p[:, :_CODE_DIM]
    cq = cq_p[:, :_CODE_DIM]
    loss = jnp.sum(lp[:, 0, 0]) * ((1.0 + _BETA) / (_B * _CODE_DIM))
    return (q, ct, ch, cq, loss)


# speedup vs baseline: 1.0616x; 1.0616x over previous
import functools

import jax
import jax.numpy as jnp
from jax import lax
from jax.experimental import pallas as pl
from jax.experimental.pallas import tpu as pltpu
from jax.experimental.pallas import tpu_sc as plsc

_NUM_CODES = 8192
_CODE_DIM = 32
_BETA = 0.25
_B = 8192
_BR = 256
_NB = _B // _BR

_NC = 2
_NS = 16
_NW = _NC * _NS
_RPW = _B // _NW
_GCH = 128
_NCH = _RPW // _GCH


def _vq_body(h_ref, cbt_ref, cb_ref, g_ref, q_ref, ct_ref, idx_ref, lp_ref):
    h = h_ref[...]
    cbt2 = cbt_ref[...]
    cb_sq = 0.25 * jnp.sum(cbt2 * cbt2, axis=0, keepdims=True)
    h_sq = jnp.sum(h * h, axis=1, keepdims=True)
    prod2 = jnp.dot(h, cbt2, preferred_element_type=jnp.float32)
    dist = (h_sq + cb_sq) - prod2
    x = g_ref[...] - dist
    m = jnp.max(x, axis=1, keepdims=True)
    e = jnp.exp(x - m)
    s = jnp.sum(e, axis=1, keepdims=True)
    inv = 1.0 / s
    q = e * inv
    q_ref[...] = q

    cb16 = cb_ref[...].astype(jnp.bfloat16)
    ct = jnp.dot(q.astype(jnp.bfloat16), cb16,
                 preferred_element_type=jnp.float32)
    ct_ref[...] = ct

    iota = jax.lax.broadcasted_iota(jnp.int32, (_BR, _NUM_CODES), 1)
    idx_ref[...] = jnp.min(jnp.where(x == m, iota, _NUM_CODES),
                           axis=1, keepdims=True)
    dsel = jnp.min(jnp.where(x == m, dist, jnp.float32(3.0e38)),
                   axis=1, keepdims=True)
    part = jnp.sum(dsel, axis=(0, 1), keepdims=True)
    lp_ref[...] = jnp.broadcast_to(part[None], (1, 1, 128))


_DPAD = 128


def _sc_gather(cb_hbm, idx_hbm, ch_hbm, cq_hbm, idx_v, rows_v, sem):
    wid = lax.axis_index("s") * _NC + lax.axis_index("c")
    base = wid * _RPW
    pltpu.sync_copy(idx_hbm.at[pl.ds(wid * _NCH, _NCH)], idx_v)
    for j in range(_NCH):
        pltpu.async_copy(cb_hbm.at[idx_v.at[j]],
                         rows_v.at[pl.ds(j * _GCH, _GCH)], sem).wait()
    pltpu.sync_copy(rows_v, ch_hbm.at[pl.ds(base, _RPW)])
    pltpu.sync_copy(rows_v, cq_hbm.at[pl.ds(base, _RPW)])


@jax.jit
def kernel(h, codebook, gumbel):
    cbt2 = 2.0 * codebook.T
    q, ct, idx, lp = pl.pallas_call(
        _vq_body,
        grid=(_NB,),
        in_specs=[
            pl.BlockSpec((_BR, _CODE_DIM), lambda i: (i, 0)),
            pl.BlockSpec((_CODE_DIM, _NUM_CODES), lambda i: (0, 0)),
            pl.BlockSpec((_NUM_CODES, _CODE_DIM), lambda i: (0, 0)),
            pl.BlockSpec((_BR, _NUM_CODES), lambda i: (i, 0)),
        ],
        out_specs=[
            pl.BlockSpec((_BR, _NUM_CODES), lambda i: (i, 0)),
            pl.BlockSpec((_BR, _CODE_DIM), lambda i: (i, 0)),
            pl.BlockSpec((_BR, 1), lambda i: (i, 0)),
            pl.BlockSpec((1, 1, 128), lambda i: (i, 0, 0)),
        ],
        out_shape=[
            jax.ShapeDtypeStruct((_B, _NUM_CODES), jnp.float32),
            jax.ShapeDtypeStruct((_B, _CODE_DIM), jnp.float32),
            jax.ShapeDtypeStruct((_B, 1), jnp.int32),
            jax.ShapeDtypeStruct((_NB, 1, 128), jnp.float32),
        ],
        compiler_params=pltpu.CompilerParams(
            dimension_semantics=("parallel",),
            vmem_limit_bytes=65536 * 1024,
        ),
    )(h, cbt2, codebook, gumbel)

    idx2d = idx.reshape(_NW * _NCH, _GCH)
    cb_pad = jnp.pad(codebook, ((0, 0), (0, _DPAD - _CODE_DIM)))

    mesh = plsc.VectorSubcoreMesh(core_axis_name="c", subcore_axis_name="s")
    sc = pl.kernel(
        _sc_gather,
        mesh=mesh,
        out_type=[
            jax.ShapeDtypeStruct((_B, _DPAD), jnp.float32),
            jax.ShapeDtypeStruct((_B, _DPAD), jnp.float32),
        ],
        scratch_types=[
            pltpu.VMEM((_NCH, _GCH), jnp.int32),
            pltpu.VMEM((_RPW, _DPAD), jnp.float32),
            pltpu.SemaphoreType.DMA,
        ],
    )
    ch_p, cq_p = sc(cb_pad, idx2d)
    ch = ch_p[:, :_CODE_DIM]
    cq = cq_p[:, :_CODE_DIM]
    loss = jnp.sum(lp[:, 0, 0]) * ((1.0 + _BETA) / (_B * _CODE_DIM))
    return (q, ct, ch, cq, loss)


# speedup vs baseline: 1.1637x; 1.0962x over previous
import functools

import jax
import jax.numpy as jnp
from jax import lax
from jax.experimental import pallas as pl
from jax.experimental.pallas import tpu as pltpu
from jax.experimental.pallas import tpu_sc as plsc

_NUM_CODES = 8192
_CODE_DIM = 32
_BETA = 0.25
_B = 8192
_BR = 256
_NB = _B // _BR

_NC = 2
_NS = 16
_NW = _NC * _NS
_RPW = _B // _NW
_GCH = 128
_NCH = _RPW // _GCH


def _vq_body(h_ref, cbt_ref, cb_ref, g_ref, q_ref, ct_ref, idx_ref):
    h = h_ref[...]
    cbt2 = cbt_ref[...]
    cb_sq = 0.25 * jnp.sum(cbt2 * cbt2, axis=0, keepdims=True)
    h_sq = jnp.sum(h * h, axis=1, keepdims=True)
    prod2 = jnp.dot(h, cbt2, preferred_element_type=jnp.float32)
    dist = (h_sq + cb_sq) - prod2
    x = g_ref[...] - dist
    m = jnp.max(x, axis=1, keepdims=True)
    e = jnp.exp(x - m)
    s = jnp.sum(e, axis=1, keepdims=True)
    inv = 1.0 / s
    q = e * inv
    q_ref[...] = q

    cb16 = cb_ref[...].astype(jnp.bfloat16)
    ct = jnp.dot(q.astype(jnp.bfloat16), cb16,
                 preferred_element_type=jnp.float32)
    ct_ref[...] = ct

    iota = jax.lax.broadcasted_iota(jnp.int32, (_BR, _NUM_CODES), 1)
    idx_ref[...] = jnp.min(jnp.where(x == m, iota, _NUM_CODES),
                           axis=1, keepdims=True)


_DPAD = 128


_NSLOT = 4


def _sc_gather(cb_hbm, idx_hbm, h_hbm, ch_hbm, cq_hbm, lp_hbm,
               idx_v, rows_v, h_v, acc_v, sem):
    wid = lax.axis_index("s") * _NC + lax.axis_index("c")
    base = wid * _RPW
    pltpu.sync_copy(idx_hbm.at[pl.ds(wid * _NCH, _NCH)], idx_v)
    for j in range(_NCH):
        pltpu.async_copy(cb_hbm.at[idx_v.at[j]],
                         rows_v.at[pl.ds(j * _GCH, _GCH)], sem).wait()
    pltpu.sync_copy(rows_v, ch_hbm.at[pl.ds(base, _RPW)])
    pltpu.sync_copy(rows_v, cq_hbm.at[pl.ds(base, _RPW)])

    pltpu.sync_copy(h_hbm.at[pl.ds(base, _RPW)], h_v)
    for t in range(_NSLOT * _CODE_DIM // 16):
        acc_v[pl.ds(t * 16, 16)] = jnp.zeros((16,), jnp.float32)

    @pl.loop(0, _RPW, step=_NSLOT)
    def _(i):
        for t in range(_NSLOT):
            for c in range(0, _CODE_DIM, 16):
                hv = h_v[i + t, pl.ds(c, 16)]
                cv = rows_v[i + t, pl.ds(c, 16)]
                d = hv - cv
                plsc.addupdate(acc_v.at[pl.ds(t * _CODE_DIM + c, 16)], d * d)

    pltpu.sync_copy(acc_v, lp_hbm.at[wid])


@jax.jit
def kernel(h, codebook, gumbel):
    cbt2 = 2.0 * codebook.T
    q, ct, idx = pl.pallas_call(
        _vq_body,
        grid=(_NB,),
        in_specs=[
            pl.BlockSpec((_BR, _CODE_DIM), lambda i: (i, 0)),
            pl.BlockSpec((_CODE_DIM, _NUM_CODES), lambda i: (0, 0)),
            pl.BlockSpec((_NUM_CODES, _CODE_DIM), lambda i: (0, 0)),
            pl.BlockSpec((_BR, _NUM_CODES), lambda i: (i, 0)),
        ],
        out_specs=[
            pl.BlockSpec((_BR, _NUM_CODES), lambda i: (i, 0)),
            pl.BlockSpec((_BR, _CODE_DIM), lambda i: (i, 0)),
            pl.BlockSpec((_BR, 1), lambda i: (i, 0)),
        ],
        out_shape=[
            jax.ShapeDtypeStruct((_B, _NUM_CODES), jnp.float32),
            jax.ShapeDtypeStruct((_B, _CODE_DIM), jnp.float32),
            jax.ShapeDtypeStruct((_B, 1), jnp.int32),
        ],
        compiler_params=pltpu.CompilerParams(
            dimension_semantics=("parallel",),
        ),
    )(h, cbt2, codebook, gumbel)

    idx2d = idx.reshape(_NW * _NCH, _GCH)
    cb_pad = jnp.pad(codebook, ((0, 0), (0, _DPAD - _CODE_DIM)))

    mesh = plsc.VectorSubcoreMesh(core_axis_name="c", subcore_axis_name="s")
    sc = pl.kernel(
        _sc_gather,
        mesh=mesh,
        out_type=[
            jax.ShapeDtypeStruct((_B, _DPAD), jnp.float32),
            jax.ShapeDtypeStruct((_B, _DPAD), jnp.float32),
            jax.ShapeDtypeStruct((_NW, _NSLOT * _CODE_DIM), jnp.float32),
        ],
        scratch_types=[
            pltpu.VMEM((_NCH, _GCH), jnp.int32),
            pltpu.VMEM((_RPW, _DPAD), jnp.float32),
            pltpu.VMEM((_RPW, _CODE_DIM), jnp.float32),
            pltpu.VMEM((_NSLOT * _CODE_DIM,), jnp.float32),
            pltpu.SemaphoreType.DMA,
        ],
    )
    ch_p, cq_p, lp = sc(cb_pad, idx2d, h)
    ch = ch_p[:, :_CODE_DIM]
    cq = cq_p[:, :_CODE_DIM]
    loss = jnp.sum(lp) * ((1.0 + _BETA) / (_B * _CODE_DIM))
    return (q, ct, ch, cq, loss)
